# baseline (speedup 1.0000x reference)
import functools

import jax
import jax.numpy as jnp
from jax import lax
from jax.experimental import pallas as pl
from jax.experimental.pallas import tpu as pltpu
from jax.experimental.pallas import tpu_sc as plsc

ND = 10000
ED = 160000
DM = 128
NC = 2
NS = 16
NW = NC * NS
CH = 128
NCH = ED // CH
MAXCH_W = (NCH + NW - 1) // NW
NDP = 10240
RPT = NDP // NS

DMW = DM // 2

NBLK = 1000
EBLK = 1600


def _silu(x):
    return x * (1.0 / (1.0 + jnp.exp(-x)))


def _ln(h, g, b):
    m = jnp.mean(h, axis=-1, keepdims=True)
    v = jnp.mean((h - m) ** 2, axis=-1, keepdims=True)
    return (h - m) * lax.rsqrt(v + 1e-5) * g + b


def _prep_body(xs_ref, xd_ref, wi_ref, wj_ref, nw1s_ref, nb1_ref, nw2_ref,
               nb2_ref, g_ref, b_ref, pd_ref, ps_ref, ns_ref):
    xs = xs_ref[...]
    xd = xd_ref[...]
    pd_ref[...] = jnp.dot(xd, wi_ref[...],
                          preferred_element_type=jnp.float32).astype(jnp.bfloat16)
    ps_ref[...] = jnp.dot(xs, wj_ref[...],
                          preferred_element_type=jnp.float32).astype(jnp.bfloat16)
    z = jnp.dot(xs, nw1s_ref[...], preferred_element_type=jnp.float32) + nb1_ref[...]
    h = jnp.dot(_silu(z), nw2_ref[...], preferred_element_type=jnp.float32) + nb2_ref[...]
    ns_ref[...] = _ln(h, g_ref[...], b_ref[...]) + xs


def _edge_body(za_ref, zb_ref, ea_ref, we_ref, b1_ref, w2_ref, b2_ref, g_ref,
               b_ref, out_ref):
    ea = ea_ref[...]
    z = (za_ref[...].astype(jnp.float32) + zb_ref[...].astype(jnp.float32)
         + jnp.dot(ea.astype(jnp.bfloat16), we_ref[...],
                   preferred_element_type=jnp.float32) + b1_ref[...])
    h = jnp.dot(_silu(z).astype(jnp.bfloat16), w2_ref[...],
                preferred_element_type=jnp.float32) + b2_ref[...]
    out_ref[...] = _ln(h, g_ref[...], b_ref[...]) + ea


def _ndst_body(xd_ref, p0_ref, p1_ref, w1a_ref, w1b_ref, nb1_ref, nw2_ref,
               nb2_ref, g_ref, b_ref, out_ref):
    xd = xd_ref[...]
    agg = p0_ref[...] + p1_ref[...]
    z = (jnp.dot(xd, w1a_ref[...], preferred_element_type=jnp.float32)
         + jnp.dot(agg, w1b_ref[...], preferred_element_type=jnp.float32)
         + nb1_ref[...])
    h = jnp.dot(_silu(z), nw2_ref[...], preferred_element_type=jnp.float32) + nb2_ref[...]
    out_ref[...] = _ln(h, g_ref[...], b_ref[...]) + xd


def _row_spec(blk):
    return pl.BlockSpec((blk, DM), lambda i: (i, 0))


def _w_spec():
    return pl.BlockSpec((DM, DM), lambda i: (0, 0))


def _b_spec():
    return pl.BlockSpec((1, DM), lambda i: (0, 0))


_prep_call = pl.pallas_call(
    _prep_body,
    grid=(ND // NBLK,),
    in_specs=[_row_spec(NBLK), _row_spec(NBLK),
              _w_spec(), _w_spec(), _w_spec(), _b_spec(), _w_spec(),
              _b_spec(), _b_spec(), _b_spec()],
    out_specs=[_row_spec(NBLK), _row_spec(NBLK), _row_spec(NBLK)],
    out_shape=[jax.ShapeDtypeStruct((ND, DM), jnp.bfloat16),
               jax.ShapeDtypeStruct((ND, DM), jnp.bfloat16),
               jax.ShapeDtypeStruct((ND, DM), jnp.float32)],
)

@functools.cache
def _edge_call(ne, blk_off):
    ea_spec = pl.BlockSpec((EBLK, DM), lambda i: (i + blk_off, 0))
    return pl.pallas_call(
        _edge_body,
        grid=(ne // EBLK,),
        in_specs=[_row_spec(EBLK), _row_spec(EBLK), ea_spec,
                  _w_spec(), _b_spec(), _w_spec(), _b_spec(), _b_spec(),
                  _b_spec()],
        out_specs=_row_spec(EBLK),
        out_shape=jax.ShapeDtypeStruct((ne, DM), jnp.float32),
    )

_ndst_call = pl.pallas_call(
    _ndst_body,
    grid=(ND // NBLK,),
    in_specs=[_row_spec(NBLK), _row_spec(NBLK), _row_spec(NBLK),
              _w_spec(), _w_spec(), _b_spec(), _w_spec(), _b_spec(),
              _b_spec(), _b_spec()],
    out_specs=_row_spec(NBLK),
    out_shape=jax.ShapeDtypeStruct((ND, DM), jnp.float32),
)


def _make_gather_body(nch, ebase):
    def gather_body(pd_hbm, ps_hbm, src_hbm, dst_hbm, za_hbm, zb_hbm,
                    sidx0, sidx1, didx0, didx1, ra0, ra1, rb0, rb1,
                    si0, si1, sga0, sga1, sgb0, sgb1):
        cid = lax.axis_index("c")
        sid = lax.axis_index("s")
        wid = sid * NC + cid
        sidx = (sidx0, sidx1)
        didx = (didx0, didx1)
        ra = (ra0, ra1)
        rb = (rb0, rb1)
        si = (si0, si1)
        sga = (sga0, sga1)
        sgb = (sgb0, sgb1)

        def chunk_of(g):
            return g * NW + wid

        def idx_start(g, p):
            base = ebase + chunk_of(g) * CH
            pltpu.async_copy(src_hbm.at[pl.ds(base, CH)], sidx[p], si[p])
            pltpu.async_copy(dst_hbm.at[pl.ds(base, CH)], didx[p], si[p])

        def idx_wait(g, p):
            base = ebase + chunk_of(g) * CH
            pltpu.make_async_copy(src_hbm.at[pl.ds(base, CH)], sidx[p], si[p]).wait()
            pltpu.make_async_copy(dst_hbm.at[pl.ds(base, CH)], didx[p], si[p]).wait()

        def gather_start(p):
            pltpu.async_copy(ps_hbm.at[sidx[p]], ra[p], sga[p])
            pltpu.async_copy(pd_hbm.at[didx[p]], rb[p], sgb[p])

        def gather_wait(p):
            pltpu.make_async_copy(ps_hbm.at[sidx[p]], ra[p], sga[p]).wait()
            pltpu.make_async_copy(pd_hbm.at[didx[p]], rb[p], sgb[p]).wait()

        idx_start(0, 0)
        idx_wait(0, 0)
        gather_start(0)
        idx_start(1, 1)

        def iter_body(i, carry):
            for k in (0, 1):
                g = 2 * i + k
                p = k
                q = 1 - k
                gv = chunk_of(g) < nch

                @pl.when(gv)
                def _():
                    gather_wait(p)

                @pl.when(chunk_of(g + 1) < nch)
                def _():
                    idx_wait(g + 1, q)
                    gather_start(q)

                @pl.when(chunk_of(g + 2) < nch)
                def _():
                    idx_start(g + 2, p)

                @pl.when(gv)
                def _():
                    base = chunk_of(g) * CH
                    pltpu.sync_copy(ra[p], za_hbm.at[pl.ds(base, CH)])
                    pltpu.sync_copy(rb[p], zb_hbm.at[pl.ds(base, CH)])

            return carry

        maxch = (nch + NW - 1) // NW
        lax.fori_loop(0, (maxch + 1) // 2, iter_body, 0)

    return gather_body


@functools.cache
def _sc_mesh():
    return plsc.VectorSubcoreMesh(core_axis_name="c", subcore_axis_name="s",
                                  num_cores=NC, num_subcores=NS)


@functools.cache
def _gather_call(ne, ebase):
    return functools.partial(
        pl.kernel,
        out_type=[jax.ShapeDtypeStruct((ne, DMW), jnp.int32),
                  jax.ShapeDtypeStruct((ne, DMW), jnp.int32)],
        mesh=_sc_mesh(),
        compiler_params=pltpu.CompilerParams(use_tc_tiling_on_sc=False),
        scratch_types=[
            pltpu.VMEM((CH,), jnp.int32),
            pltpu.VMEM((CH,), jnp.int32),
            pltpu.VMEM((CH,), jnp.int32),
            pltpu.VMEM((CH,), jnp.int32),
            pltpu.VMEM((CH, DMW), jnp.int32),
            pltpu.VMEM((CH, DMW), jnp.int32),
            pltpu.VMEM((CH, DMW), jnp.int32),
            pltpu.VMEM((CH, DMW), jnp.int32),
            pltpu.SemaphoreType.DMA,
            pltpu.SemaphoreType.DMA,
            pltpu.SemaphoreType.DMA,
            pltpu.SemaphoreType.DMA,
            pltpu.SemaphoreType.DMA,
            pltpu.SemaphoreType.DMA,
        ],
    )(_make_gather_body(ne // CH, ebase))


NW_SC = NS
MAXCH_SC = (NCH + NW_SC - 1) // NW_SC


def _make_scatter_body(nch, ebase):
    def scatter_body(en_hbm, dst_hbm, out_hbm, idx0, idx1, rows0, rows1,
                     si0, si1, sr0, sr1, sc0, sc1, acc):
        sid = lax.axis_index("s")
        wid = sid
        idx = (idx0, idx1)
        rows = (rows0, rows1)
        si = (si0, si1)
        sr = (sr0, sr1)
        sc = (sc0, sc1)

        def zrow(r, c):
            for j in range(DM // 16):
                rows0[r, pl.ds(j * 16, 16)] = jnp.zeros((16,), jnp.float32)
            return c

        lax.fori_loop(0, CH, zrow, 0)
        for k in range(RPT // CH):
            pltpu.sync_copy(rows0, acc.at[pl.ds(sid * RPT + k * CH, CH)])
        plsc.subcore_barrier()

        def chunk_of(g):
            return g * NW_SC + wid

        def read_start(g, p):
            base = chunk_of(g) * CH
            pltpu.async_copy(dst_hbm.at[pl.ds(ebase + base, CH)], idx[p], si[p])
            pltpu.async_copy(en_hbm.at[pl.ds(base, CH)], rows[p], sr[p])

        def read_wait(g, p):
            base = chunk_of(g) * CH
            pltpu.make_async_copy(dst_hbm.at[pl.ds(ebase + base, CH)],
                                  idx[p], si[p]).wait()
            pltpu.make_async_copy(en_hbm.at[pl.ds(base, CH)],
                                  rows[p], sr[p]).wait()

        read_start(0, 0)

        def body(i, c):
            for k in (0, 1):
                g = 2 * i + k
                p = k
                q = 1 - k

                prev_ok = chunk_of(g - 1) < nch
                if k == 0:
                    prev_ok = (i > 0) & prev_ok

                @pl.when(prev_ok)
                def _():
                    pltpu.make_async_copy(rows[q], acc.at[idx[q]], sc[q]).wait()

                gv = chunk_of(g) < nch

                @pl.when(gv & (chunk_of(g + 1) < nch))
                def _():
                    read_start(g + 1, q)

                @pl.when(gv)
                def _():
                    read_wait(g, p)
                    pltpu.async_copy(rows[p], acc.at[idx[p]], sc[p], add=True)

            return c

        maxch = (nch + NW_SC - 1) // NW_SC
        lax.fori_loop(0, (maxch + 1) // 2 + 1, body, 0)
        plsc.subcore_barrier()
        pltpu.sync_copy(acc.at[pl.ds(sid * RPT, RPT)],
                        out_hbm.at[pl.ds(sid * RPT, RPT)])

    return scatter_body


@functools.cache
def _scatter_mesh():
    return plsc.VectorSubcoreMesh(core_axis_name="c", subcore_axis_name="s",
                                  num_cores=1, num_subcores=NS)


@functools.cache
def _scatter_call(ne, ebase):
    return functools.partial(
        pl.kernel,
        out_type=jax.ShapeDtypeStruct((NDP, DM), jnp.float32),
        mesh=_scatter_mesh(),
        scratch_types=[
            pltpu.VMEM((CH,), jnp.int32),
            pltpu.VMEM((CH,), jnp.int32),
            pltpu.VMEM((CH, DM), jnp.float32),
            pltpu.VMEM((CH, DM), jnp.float32),
            pltpu.SemaphoreType.DMA,
            pltpu.SemaphoreType.DMA,
            pltpu.SemaphoreType.DMA,
            pltpu.SemaphoreType.DMA,
            pltpu.SemaphoreType.DMA,
            pltpu.SemaphoreType.DMA,
            pltpu.VMEM_SHARED((NDP, DM), jnp.float32),
        ],
    )(_make_scatter_body(ne // CH, ebase))


def kernel(x_src, x_dst, edge_attr, edge_index,
           conv_w1, conv_b1, conv_w2, conv_b2, conv_ln_g, conv_ln_b,
           node_w1, node_b1, node_w2, node_b2, node_ln_g, node_ln_b):
    src = edge_index[0]
    dst = edge_index[1]
    wi = conv_w1[:DM]
    wj = conv_w1[DM:2 * DM]
    we = conv_w1[2 * DM:]
    nw1a = node_w1[:DM]
    nw1b = node_w1[DM:]
    nw1s = nw1a + nw1b
    cb1 = conv_b1.reshape(1, DM)
    cb2 = conv_b2.reshape(1, DM)
    cg = conv_ln_g.reshape(1, DM)
    cb = conv_ln_b.reshape(1, DM)
    nb1 = node_b1.reshape(1, DM)
    nb2 = node_b2.reshape(1, DM)
    ng = node_ln_g.reshape(1, DM)
    nb = node_ln_b.reshape(1, DM)

    pd, ps, nodes_new_src = _prep_call(
        x_src, x_dst, wi, wj, nw1s, nb1, node_w2, nb2, ng, nb)

    e2 = ED // 2
    web = we.astype(jnp.bfloat16)
    w2b = conv_w2.astype(jnp.bfloat16)
    pdw = lax.bitcast_convert_type(pd.reshape(ND, DMW, 2), jnp.int32)
    psw = lax.bitcast_convert_type(ps.reshape(ND, DMW, 2), jnp.int32)
    z0aw, z0bw = _gather_call(e2, 0)(pdw, psw, src, dst)
    z1aw, z1bw = _gather_call(e2, e2)(pdw, psw, src, dst)

    def _unpack(zw):
        return lax.bitcast_convert_type(zw, jnp.bfloat16).reshape(e2, DM)

    en0 = _edge_call(e2, 0)(_unpack(z0aw), _unpack(z0bw), edge_attr,
                            web, cb1, w2b, cb2, cg, cb)
    en1 = _edge_call(e2, e2 // EBLK)(_unpack(z1aw), _unpack(z1bw), edge_attr,
                                     web, cb1, w2b, cb2, cg, cb)
    agg0 = _scatter_call(e2, 0)(en0, dst)
    agg1 = _scatter_call(e2, e2)(en1, dst)
    edges_new = jnp.concatenate([en0, en1], axis=0)
    nodes_new_dst = _ndst_call(
        x_dst, agg0, agg1, nw1a, nw1b, nb1, node_w2, nb2, ng, nb)
    return nodes_new_src, nodes_new_dst, edges_new


# speedup vs baseline: 3.6118x; 3.6118x over previous
import functools

import jax
import jax.numpy as jnp
from jax import lax
from jax.experimental import pallas as pl
from jax.experimental.pallas import tpu as pltpu
from jax.experimental.pallas import tpu_sc as plsc

ND = 10000
ED = 160000
DM = 128
NC = 2
NS = 16
NW = NC * NS
CH = 128
NCH = ED // CH
MAXCH_W = (NCH + NW - 1) // NW
NDP = 10240
RPT = NDP // NS

DMW = DM // 2

NBLK = 1000
EBLK = 1600


def _silu(x):
    return x * (1.0 / (1.0 + jnp.exp(-x)))


def _ln(h, g, b):
    m = jnp.mean(h, axis=-1, keepdims=True)
    v = jnp.mean((h - m) ** 2, axis=-1, keepdims=True)
    return (h - m) * lax.rsqrt(v + 1e-5) * g + b


def _prep_body(xs_ref, xd_ref, wi_ref, wj_ref, nw1s_ref, nb1_ref, nw2_ref,
               nb2_ref, g_ref, b_ref, pd_ref, ps_ref, ns_ref):
    xs = xs_ref[...]
    xd = xd_ref[...]
    pd_ref[...] = jnp.dot(xd, wi_ref[...], preferred_element_type=jnp.float32)
    ps_ref[...] = jnp.dot(xs, wj_ref[...], preferred_element_type=jnp.float32)
    z = jnp.dot(xs, nw1s_ref[...], preferred_element_type=jnp.float32) + nb1_ref[...]
    h = jnp.dot(_silu(z), nw2_ref[...], preferred_element_type=jnp.float32) + nb2_ref[...]
    ns_ref[...] = _ln(h, g_ref[...], b_ref[...]) + xs


def _edge_body(z_ref, ea_ref, we_ref, b1_ref, w2_ref, b2_ref, g_ref,
               b_ref, out_ref):
    ea = ea_ref[...]
    z = (z_ref[...]
         + jnp.dot(ea.astype(jnp.bfloat16), we_ref[...],
                   preferred_element_type=jnp.float32) + b1_ref[...])
    h = jnp.dot(_silu(z).astype(jnp.bfloat16), w2_ref[...],
                preferred_element_type=jnp.float32) + b2_ref[...]
    out_ref[...] = _ln(h, g_ref[...], b_ref[...]) + ea


def _ndst_body(xd_ref, p0_ref, p1_ref, w1a_ref, w1b_ref, nb1_ref, nw2_ref,
               nb2_ref, g_ref, b_ref, out_ref):
    xd = xd_ref[...]
    agg = p0_ref[...] + p1_ref[...]
    z = (jnp.dot(xd, w1a_ref[...], preferred_element_type=jnp.float32)
         + jnp.dot(agg, w1b_ref[...], preferred_element_type=jnp.float32)
         + nb1_ref[...])
    h = jnp.dot(_silu(z), nw2_ref[...], preferred_element_type=jnp.float32) + nb2_ref[...]
    out_ref[...] = _ln(h, g_ref[...], b_ref[...]) + xd


def _row_spec(blk):
    return pl.BlockSpec((blk, DM), lambda i: (i, 0))


def _w_spec():
    return pl.BlockSpec((DM, DM), lambda i: (0, 0))


def _b_spec():
    return pl.BlockSpec((1, DM), lambda i: (0, 0))


_prep_call = pl.pallas_call(
    _prep_body,
    grid=(ND // NBLK,),
    in_specs=[_row_spec(NBLK), _row_spec(NBLK),
              _w_spec(), _w_spec(), _w_spec(), _b_spec(), _w_spec(),
              _b_spec(), _b_spec(), _b_spec()],
    out_specs=[_row_spec(NBLK), _row_spec(NBLK), _row_spec(NBLK)],
    out_shape=[jax.ShapeDtypeStruct((ND, DM), jnp.float32)] * 3,
)

@functools.cache
def _edge_call(ne, blk_off):
    ea_spec = pl.BlockSpec((EBLK, DM), lambda i: (i + blk_off, 0))
    return pl.pallas_call(
        _edge_body,
        grid=(ne // EBLK,),
        in_specs=[_row_spec(EBLK), ea_spec,
                  _w_spec(), _b_spec(), _w_spec(), _b_spec(), _b_spec(),
                  _b_spec()],
        out_specs=_row_spec(EBLK),
        out_shape=jax.ShapeDtypeStruct((ne, DM), jnp.float32),
    )

_ndst_call = pl.pallas_call(
    _ndst_body,
    grid=(ND // NBLK,),
    in_specs=[_row_spec(NBLK), _row_spec(NBLK), _row_spec(NBLK),
              _w_spec(), _w_spec(), _b_spec(), _w_spec(), _b_spec(),
              _b_spec(), _b_spec()],
    out_specs=_row_spec(NBLK),
    out_shape=jax.ShapeDtypeStruct((ND, DM), jnp.float32),
)


def _make_gather_body(nch, ebase):
    def gather_body(pd_hbm, ps_hbm, src_hbm, dst_hbm, z_hbm,
                    sidx0, sidx1, didx0, didx1, ra0, ra1, rb0, rb1,
                    si0, si1, sga0, sga1, sgb0, sgb1):
        cid = lax.axis_index("c")
        sid = lax.axis_index("s")
        wid = sid * NC + cid
        sidx = (sidx0, sidx1)
        didx = (didx0, didx1)
        ra = (ra0, ra1)
        rb = (rb0, rb1)
        si = (si0, si1)
        sga = (sga0, sga1)
        sgb = (sgb0, sgb1)

        def chunk_of(g):
            return g * NW + wid

        def idx_start(g, p):
            base = ebase + chunk_of(g) * CH
            pltpu.async_copy(src_hbm.at[pl.ds(base, CH)], sidx[p], si[p])
            pltpu.async_copy(dst_hbm.at[pl.ds(base, CH)], didx[p], si[p])

        def idx_wait(g, p):
            base = ebase + chunk_of(g) * CH
            pltpu.make_async_copy(src_hbm.at[pl.ds(base, CH)], sidx[p], si[p]).wait()
            pltpu.make_async_copy(dst_hbm.at[pl.ds(base, CH)], didx[p], si[p]).wait()

        def gather_start(p):
            pltpu.async_copy(ps_hbm.at[sidx[p]], ra[p], sga[p])
            pltpu.async_copy(pd_hbm.at[didx[p]], rb[p], sgb[p])

        def gather_wait(p):
            pltpu.make_async_copy(ps_hbm.at[sidx[p]], ra[p], sga[p]).wait()
            pltpu.make_async_copy(pd_hbm.at[didx[p]], rb[p], sgb[p]).wait()

        idx_start(0, 0)
        idx_wait(0, 0)
        gather_start(0)
        idx_start(1, 1)

        def iter_body(i, carry):
            for k in (0, 1):
                g = 2 * i + k
                p = k
                q = 1 - k
                gv = chunk_of(g) < nch

                @pl.when(gv)
                def _():
                    gather_wait(p)

                @pl.when(chunk_of(g + 1) < nch)
                def _():
                    idx_wait(g + 1, q)
                    gather_start(q)

                @pl.when(chunk_of(g + 2) < nch)
                def _():
                    idx_start(g + 2, p)

                @pl.when(gv)
                def _():
                    def add_row(r, c2):
                        for j in range(DM // 16):
                            sl = pl.ds(j * 16, 16)
                            ra[p][r, sl] = ra[p][r, sl] + rb[p][r, sl]
                        return c2

                    lax.fori_loop(0, CH, add_row, 0)
                    pltpu.sync_copy(ra[p], z_hbm.at[pl.ds(chunk_of(g) * CH, CH)])

            return carry

        maxch = (nch + NW - 1) // NW
        lax.fori_loop(0, (maxch + 1) // 2, iter_body, 0)

    return gather_body


@functools.cache
def _sc_mesh():
    return plsc.VectorSubcoreMesh(core_axis_name="c", subcore_axis_name="s",
                                  num_cores=NC, num_subcores=NS)


@functools.cache
def _gather_call(ne, ebase):
    return functools.partial(
        pl.kernel,
        out_type=jax.ShapeDtypeStruct((ne, DM), jnp.float32),
        mesh=_sc_mesh(),
        scratch_types=[
            pltpu.VMEM((CH,), jnp.int32),
            pltpu.VMEM((CH,), jnp.int32),
            pltpu.VMEM((CH,), jnp.int32),
            pltpu.VMEM((CH,), jnp.int32),
            pltpu.VMEM((CH, DM), jnp.float32),
            pltpu.VMEM((CH, DM), jnp.float32),
            pltpu.VMEM((CH, DM), jnp.float32),
            pltpu.VMEM((CH, DM), jnp.float32),
            pltpu.SemaphoreType.DMA,
            pltpu.SemaphoreType.DMA,
            pltpu.SemaphoreType.DMA,
            pltpu.SemaphoreType.DMA,
            pltpu.SemaphoreType.DMA,
            pltpu.SemaphoreType.DMA,
        ],
    )(_make_gather_body(ne // CH, ebase))


NW_SC = NS
MAXCH_SC = (NCH + NW_SC - 1) // NW_SC


def _make_scatter_body(nch, ebase):
    def scatter_body(en_hbm, dst_hbm, out_hbm, idx0, idx1, rows0, rows1,
                     si0, si1, sr0, sr1, sc0, sc1, acc):
        sid = lax.axis_index("s")
        wid = sid
        idx = (idx0, idx1)
        rows = (rows0, rows1)
        si = (si0, si1)
        sr = (sr0, sr1)
        sc = (sc0, sc1)

        def zrow(r, c):
            for j in range(DM // 16):
                rows0[r, pl.ds(j * 16, 16)] = jnp.zeros((16,), jnp.float32)
            return c

        lax.fori_loop(0, CH, zrow, 0)
        for k in range(RPT // CH):
            pltpu.sync_copy(rows0, acc.at[pl.ds(sid * RPT + k * CH, CH)])
        plsc.subcore_barrier()

        def chunk_of(g):
            return g * NW_SC + wid

        def read_start(g, p):
            base = chunk_of(g) * CH
            pltpu.async_copy(dst_hbm.at[pl.ds(ebase + base, CH)], idx[p], si[p])
            pltpu.async_copy(en_hbm.at[pl.ds(base, CH)], rows[p], sr[p])

        def read_wait(g, p):
            base = chunk_of(g) * CH
            pltpu.make_async_copy(dst_hbm.at[pl.ds(ebase + base, CH)],
                                  idx[p], si[p]).wait()
            pltpu.make_async_copy(en_hbm.at[pl.ds(base, CH)],
                                  rows[p], sr[p]).wait()

        read_start(0, 0)

        def body(i, c):
            for k in (0, 1):
                g = 2 * i + k
                p = k
                q = 1 - k

                prev_ok = chunk_of(g - 1) < nch
                if k == 0:
                    prev_ok = (i > 0) & prev_ok

                @pl.when(prev_ok)
                def _():
                    pltpu.make_async_copy(rows[q], acc.at[idx[q]], sc[q]).wait()

                gv = chunk_of(g) < nch

                @pl.when(gv & (chunk_of(g + 1) < nch))
                def _():
                    read_start(g + 1, q)

                @pl.when(gv)
                def _():
                    read_wait(g, p)
                    pltpu.async_copy(rows[p], acc.at[idx[p]], sc[p], add=True)

            return c

        maxch = (nch + NW_SC - 1) // NW_SC
        lax.fori_loop(0, (maxch + 1) // 2 + 1, body, 0)
        plsc.subcore_barrier()
        pltpu.sync_copy(acc.at[pl.ds(sid * RPT, RPT)],
                        out_hbm.at[pl.ds(sid * RPT, RPT)])

    return scatter_body


@functools.cache
def _scatter_mesh():
    return plsc.VectorSubcoreMesh(core_axis_name="c", subcore_axis_name="s",
                                  num_cores=1, num_subcores=NS)


@functools.cache
def _scatter_call(ne, ebase):
    return functools.partial(
        pl.kernel,
        out_type=jax.ShapeDtypeStruct((NDP, DM), jnp.float32),
        mesh=_scatter_mesh(),
        scratch_types=[
            pltpu.VMEM((CH,), jnp.int32),
            pltpu.VMEM((CH,), jnp.int32),
            pltpu.VMEM((CH, DM), jnp.float32),
            pltpu.VMEM((CH, DM), jnp.float32),
            pltpu.SemaphoreType.DMA,
            pltpu.SemaphoreType.DMA,
            pltpu.SemaphoreType.DMA,
            pltpu.SemaphoreType.DMA,
            pltpu.SemaphoreType.DMA,
            pltpu.SemaphoreType.DMA,
            pltpu.VMEM_SHARED((NDP, DM), jnp.float32),
        ],
    )(_make_scatter_body(ne // CH, ebase))


def kernel(x_src, x_dst, edge_attr, edge_index,
           conv_w1, conv_b1, conv_w2, conv_b2, conv_ln_g, conv_ln_b,
           node_w1, node_b1, node_w2, node_b2, node_ln_g, node_ln_b):
    src = edge_index[0]
    dst = edge_index[1]
    wi = conv_w1[:DM]
    wj = conv_w1[DM:2 * DM]
    we = conv_w1[2 * DM:]
    nw1a = node_w1[:DM]
    nw1b = node_w1[DM:]
    nw1s = nw1a + nw1b
    cb1 = conv_b1.reshape(1, DM)
    cb2 = conv_b2.reshape(1, DM)
    cg = conv_ln_g.reshape(1, DM)
    cb = conv_ln_b.reshape(1, DM)
    nb1 = node_b1.reshape(1, DM)
    nb2 = node_b2.reshape(1, DM)
    ng = node_ln_g.reshape(1, DM)
    nb = node_ln_b.reshape(1, DM)

    pd, ps, nodes_new_src = _prep_call(
        x_src, x_dst, wi, wj, nw1s, nb1, node_w2, nb2, ng, nb)

    e2 = ED // 2
    web = we.astype(jnp.bfloat16)
    w2b = conv_w2.astype(jnp.bfloat16)
    z0 = _gather_call(e2, 0)(pd, ps, src, dst)
    z1 = _gather_call(e2, e2)(pd, ps, src, dst)
    en0 = _edge_call(e2, 0)(z0, edge_attr, web, cb1, w2b, cb2, cg, cb)
    en1 = _edge_call(e2, e2 // EBLK)(z1, edge_attr, web, cb1, w2b, cb2, cg, cb)
    agg0 = _scatter_call(e2, 0)(en0, dst)
    agg1 = _scatter_call(e2, e2)(en1, dst)
    edges_new = jnp.concatenate([en0, en1], axis=0)
    nodes_new_dst = _ndst_call(
        x_dst, agg0, agg1, nw1a, nw1b, nb1, node_w2, nb2, ng, nb)
    return nodes_new_src, nodes_new_dst, edges_new


# speedup vs baseline: 3.6161x; 1.0012x over previous
import functools

import jax
import jax.numpy as jnp
from jax import lax
from jax.experimental import pallas as pl
from jax.experimental.pallas import tpu as pltpu
from jax.experimental.pallas import tpu_sc as plsc

ND = 10000
ED = 160000
DM = 128
NC = 2
NS = 16
NW = NC * NS
CH = 128
NCH = ED // CH
MAXCH_W = (NCH + NW - 1) // NW
NDP = 10240
RPT = NDP // NS

DMW = DM // 2

NBLK = 1000
EBLK = 1600


def _silu(x):
    return x * (1.0 / (1.0 + jnp.exp(-x)))


def _ln(h, g, b):
    m = jnp.mean(h, axis=-1, keepdims=True)
    v = jnp.mean((h - m) ** 2, axis=-1, keepdims=True)
    return (h - m) * lax.rsqrt(v + 1e-5) * g + b


def _prep_body(xs_ref, xd_ref, wi_ref, wj_ref, nw1s_ref, nb1_ref, nw2_ref,
               nb2_ref, g_ref, b_ref, pd_ref, ps_ref, ns_ref):
    xs = xs_ref[...]
    xd = xd_ref[...]
    pd_ref[...] = jnp.dot(xd, wi_ref[...], preferred_element_type=jnp.float32)
    ps_ref[...] = jnp.dot(xs, wj_ref[...], preferred_element_type=jnp.float32)
    z = jnp.dot(xs, nw1s_ref[...], preferred_element_type=jnp.float32) + nb1_ref[...]
    h = jnp.dot(_silu(z), nw2_ref[...], preferred_element_type=jnp.float32) + nb2_ref[...]
    ns_ref[...] = _ln(h, g_ref[...], b_ref[...]) + xs


def _edge_body(z_ref, ea_ref, we_ref, b1_ref, w2_ref, b2_ref, g_ref,
               b_ref, out_ref):
    ea = ea_ref[...]
    z = (z_ref[...]
         + jnp.dot(ea.astype(jnp.bfloat16), we_ref[...],
                   preferred_element_type=jnp.float32) + b1_ref[...])
    h = jnp.dot(_silu(z).astype(jnp.bfloat16), w2_ref[...],
                preferred_element_type=jnp.float32) + b2_ref[...]
    out_ref[...] = _ln(h, g_ref[...], b_ref[...]) + ea


def _ndst_body(xd_ref, p0_ref, p1_ref, w1a_ref, w1b_ref, nb1_ref, nw2_ref,
               nb2_ref, g_ref, b_ref, out_ref):
    xd = xd_ref[...]
    agg = p0_ref[...] + p1_ref[...]
    z = (jnp.dot(xd, w1a_ref[...], preferred_element_type=jnp.float32)
         + jnp.dot(agg, w1b_ref[...], preferred_element_type=jnp.float32)
         + nb1_ref[...])
    h = jnp.dot(_silu(z), nw2_ref[...], preferred_element_type=jnp.float32) + nb2_ref[...]
    out_ref[...] = _ln(h, g_ref[...], b_ref[...]) + xd


def _row_spec(blk):
    return pl.BlockSpec((blk, DM), lambda i: (i, 0))


def _w_spec():
    return pl.BlockSpec((DM, DM), lambda i: (0, 0))


def _b_spec():
    return pl.BlockSpec((1, DM), lambda i: (0, 0))


_prep_call = pl.pallas_call(
    _prep_body,
    grid=(ND // NBLK,),
    in_specs=[_row_spec(NBLK), _row_spec(NBLK),
              _w_spec(), _w_spec(), _w_spec(), _b_spec(), _w_spec(),
              _b_spec(), _b_spec(), _b_spec()],
    out_specs=[_row_spec(NBLK), _row_spec(NBLK), _row_spec(NBLK)],
    out_shape=[jax.ShapeDtypeStruct((ND, DM), jnp.float32)] * 3,
)

@functools.cache
def _edge_call(ne, blk_off):
    ea_spec = pl.BlockSpec((EBLK, DM), lambda i: (i + blk_off, 0))
    return pl.pallas_call(
        _edge_body,
        grid=(ne // EBLK,),
        in_specs=[_row_spec(EBLK), ea_spec,
                  _w_spec(), _b_spec(), _w_spec(), _b_spec(), _b_spec(),
                  _b_spec()],
        out_specs=_row_spec(EBLK),
        out_shape=jax.ShapeDtypeStruct((ne, DM), jnp.float32),
    )

_ndst_call = pl.pallas_call(
    _ndst_body,
    grid=(ND // NBLK,),
    in_specs=[_row_spec(NBLK), _row_spec(NBLK), _row_spec(NBLK),
              _w_spec(), _w_spec(), _b_spec(), _w_spec(), _b_spec(),
              _b_spec(), _b_spec()],
    out_specs=_row_spec(NBLK),
    out_shape=jax.ShapeDtypeStruct((ND, DM), jnp.float32),
)


def _make_gather_body(nch, ebase):
    def gather_body(pd_hbm, ps_hbm, src_hbm, dst_hbm, z_hbm,
                    sidx0, sidx1, didx0, didx1, ra0, ra1, rb0, rb1,
                    si0, si1, sga0, sga1, sgb0, sgb1):
        cid = lax.axis_index("c")
        sid = lax.axis_index("s")
        wid = sid * NC + cid
        sidx = (sidx0, sidx1)
        didx = (didx0, didx1)
        ra = (ra0, ra1)
        rb = (rb0, rb1)
        si = (si0, si1)
        sga = (sga0, sga1)
        sgb = (sgb0, sgb1)

        def chunk_of(g):
            return g * NW + wid

        def idx_start(g, p):
            base = ebase + chunk_of(g) * CH
            pltpu.async_copy(src_hbm.at[pl.ds(base, CH)], sidx[p], si[p])
            pltpu.async_copy(dst_hbm.at[pl.ds(base, CH)], didx[p], si[p])

        def idx_wait(g, p):
            base = ebase + chunk_of(g) * CH
            pltpu.make_async_copy(src_hbm.at[pl.ds(base, CH)], sidx[p], si[p]).wait()
            pltpu.make_async_copy(dst_hbm.at[pl.ds(base, CH)], didx[p], si[p]).wait()

        def gather_start(p):
            pltpu.async_copy(ps_hbm.at[sidx[p]], ra[p], sga[p])
            pltpu.async_copy(pd_hbm.at[didx[p]], rb[p], sgb[p])

        def gather_wait(p):
            pltpu.make_async_copy(ps_hbm.at[sidx[p]], ra[p], sga[p]).wait()
            pltpu.make_async_copy(pd_hbm.at[didx[p]], rb[p], sgb[p]).wait()

        idx_start(0, 0)
        idx_wait(0, 0)
        gather_start(0)
        idx_start(1, 1)

        def iter_body(i, carry):
            for k in (0, 1):
                g = 2 * i + k
                p = k
                q = 1 - k
                gv = chunk_of(g) < nch

                @pl.when(gv)
                def _():
                    gather_wait(p)

                @pl.when(chunk_of(g + 1) < nch)
                def _():
                    idx_wait(g + 1, q)
                    gather_start(q)

                @pl.when(chunk_of(g + 2) < nch)
                def _():
                    idx_start(g + 2, p)

                @pl.when(gv)
                def _():
                    def add_row(r, c2):
                        for j in range(DM // 16):
                            sl = pl.ds(j * 16, 16)
                            ra[p][r, sl] = ra[p][r, sl] + rb[p][r, sl]
                        return c2

                    lax.fori_loop(0, CH, add_row, 0)
                    pltpu.sync_copy(ra[p], z_hbm.at[pl.ds(chunk_of(g) * CH, CH)])

            return carry

        maxch = (nch + NW - 1) // NW
        lax.fori_loop(0, (maxch + 1) // 2, iter_body, 0)

    return gather_body


@functools.cache
def _sc_mesh():
    return plsc.VectorSubcoreMesh(core_axis_name="c", subcore_axis_name="s",
                                  num_cores=NC, num_subcores=NS)


@functools.cache
def _gather_call(ne, ebase):
    return functools.partial(
        pl.kernel,
        out_type=jax.ShapeDtypeStruct((ne, DM), jnp.float32),
        mesh=_sc_mesh(),
        scratch_types=[
            pltpu.VMEM((CH,), jnp.int32),
            pltpu.VMEM((CH,), jnp.int32),
            pltpu.VMEM((CH,), jnp.int32),
            pltpu.VMEM((CH,), jnp.int32),
            pltpu.VMEM((CH, DM), jnp.float32),
            pltpu.VMEM((CH, DM), jnp.float32),
            pltpu.VMEM((CH, DM), jnp.float32),
            pltpu.VMEM((CH, DM), jnp.float32),
            pltpu.SemaphoreType.DMA,
            pltpu.SemaphoreType.DMA,
            pltpu.SemaphoreType.DMA,
            pltpu.SemaphoreType.DMA,
            pltpu.SemaphoreType.DMA,
            pltpu.SemaphoreType.DMA,
        ],
    )(_make_gather_body(ne // CH, ebase))


NW_SC = NS


def _make_scatter_body(nch, ebase):
    def scatter_body(en_hbm, dst_hbm, out_hbm, idx0, idx1, rows0, rows1,
                     si0, si1, sr0, sr1, sc0, sc1, acc):
        sid = lax.axis_index("s")
        wid = sid
        idx = (idx0, idx1)
        rows = (rows0, rows1)
        si = (si0, si1)
        sr = (sr0, sr1)
        sc = (sc0, sc1)

        def zrow(r, c):
            for j in range(DM // 16):
                rows0[r, pl.ds(j * 16, 16)] = jnp.zeros((16,), jnp.float32)
            return c

        lax.fori_loop(0, CH, zrow, 0)
        for k in range(RPT // CH):
            pltpu.sync_copy(rows0, acc.at[pl.ds(sid * RPT + k * CH, CH)])
        plsc.subcore_barrier()

        def chunk_of(g):
            return g * NW_SC + wid

        def read_start(g, p):
            base = chunk_of(g) * CH
            pltpu.async_copy(dst_hbm.at[pl.ds(ebase + base, CH)], idx[p], si[p])
            pltpu.async_copy(en_hbm.at[pl.ds(base, CH)], rows[p], sr[p])

        def read_wait(g, p):
            base = chunk_of(g) * CH
            pltpu.make_async_copy(dst_hbm.at[pl.ds(ebase + base, CH)],
                                  idx[p], si[p]).wait()
            pltpu.make_async_copy(en_hbm.at[pl.ds(base, CH)],
                                  rows[p], sr[p]).wait()

        read_start(0, 0)

        def body(i, c):
            for k in (0, 1):
                g = 2 * i + k
                p = k
                q = 1 - k

                prev_ok = chunk_of(g - 1) < nch
                if k == 0:
                    prev_ok = (i > 0) & prev_ok

                @pl.when(prev_ok)
                def _():
                    pltpu.make_async_copy(rows[q], acc.at[idx[q]], sc[q]).wait()

                gv = chunk_of(g) < nch

                @pl.when(gv & (chunk_of(g + 1) < nch))
                def _():
                    read_start(g + 1, q)

                @pl.when(gv)
                def _():
                    read_wait(g, p)
                    pltpu.async_copy(rows[p], acc.at[idx[p]], sc[p], add=True)

            return c

        maxch = (nch + NW_SC - 1) // NW_SC
        lax.fori_loop(0, (maxch + 1) // 2 + 1, body, 0)
        plsc.subcore_barrier()
        pltpu.sync_copy(acc.at[pl.ds(sid * RPT, RPT)],
                        out_hbm.at[pl.ds(sid * RPT, RPT)])

    return scatter_body


@functools.cache
def _scatter_mesh():
    return plsc.VectorSubcoreMesh(core_axis_name="c", subcore_axis_name="s",
                                  num_cores=1, num_subcores=NS)


@functools.cache
def _scatter_call(ne, ebase):
    return functools.partial(
        pl.kernel,
        out_type=jax.ShapeDtypeStruct((NDP, DM), jnp.float32),
        mesh=_scatter_mesh(),
        scratch_types=[
            pltpu.VMEM((CH,), jnp.int32),
            pltpu.VMEM((CH,), jnp.int32),
            pltpu.VMEM((CH, DM), jnp.float32),
            pltpu.VMEM((CH, DM), jnp.float32),
            pltpu.SemaphoreType.DMA,
            pltpu.SemaphoreType.DMA,
            pltpu.SemaphoreType.DMA,
            pltpu.SemaphoreType.DMA,
            pltpu.SemaphoreType.DMA,
            pltpu.SemaphoreType.DMA,
            pltpu.VMEM_SHARED((NDP, DM), jnp.float32),
        ],
    )(_make_scatter_body(ne // CH, ebase))


def kernel(x_src, x_dst, edge_attr, edge_index,
           conv_w1, conv_b1, conv_w2, conv_b2, conv_ln_g, conv_ln_b,
           node_w1, node_b1, node_w2, node_b2, node_ln_g, node_ln_b):
    src = edge_index[0]
    dst = edge_index[1]
    wi = conv_w1[:DM]
    wj = conv_w1[DM:2 * DM]
    we = conv_w1[2 * DM:]
    nw1a = node_w1[:DM]
    nw1b = node_w1[DM:]
    nw1s = nw1a + nw1b
    cb1 = conv_b1.reshape(1, DM)
    cb2 = conv_b2.reshape(1, DM)
    cg = conv_ln_g.reshape(1, DM)
    cb = conv_ln_b.reshape(1, DM)
    nb1 = node_b1.reshape(1, DM)
    nb2 = node_b2.reshape(1, DM)
    ng = node_ln_g.reshape(1, DM)
    nb = node_ln_b.reshape(1, DM)

    pd, ps, nodes_new_src = _prep_call(
        x_src, x_dst, wi, wj, nw1s, nb1, node_w2, nb2, ng, nb)

    e2 = ED // 2
    web = we.astype(jnp.bfloat16)
    w2b = conv_w2.astype(jnp.bfloat16)
    z0 = _gather_call(e2, 0)(pd, ps, src, dst)
    z1 = _gather_call(e2, e2)(pd, ps, src, dst)
    en0 = _edge_call(e2, 0)(z0, edge_attr, web, cb1, w2b, cb2, cg, cb)
    en1 = _edge_call(e2, e2 // EBLK)(z1, edge_attr, web, cb1, w2b, cb2, cg, cb)
    agg0 = _scatter_call(e2, 0)(en0, dst)
    agg1 = _scatter_call(e2, e2)(en1, dst)
    edges_new = jnp.concatenate([en0, en1], axis=0)
    nodes_new_dst = _ndst_call(
        x_dst, agg0, agg1, nw1a, nw1b, nb1, node_w2, nb2, ng, nb)
    return nodes_new_src, nodes_new_dst, edges_new
